# baseline (speedup 1.0000x reference)
import functools

import jax
import jax.numpy as jnp
from jax import lax
from jax.experimental import pallas as pl
from jax.experimental.pallas import tpu as pltpu
from jax.experimental.pallas import tpu_sc as plsc

_N = 2048
_NW = 32
_BAND = _N // (2 * _NW)
_RPW = 2 * _BAND
_L = 16
_NJV = _N // _L
_CUTOFF = 5.0

_OX, _OY, _OZ = 0 * _N, 1 * _N, 2 * _N
_ORE, _OBE, _OAL = 3 * _N, 4 * _N, 5 * _N
_OFE, _OA, _OB = 6 * _N, 7 * _N, 8 * _N
_OKA, _OLA = 9 * _N, 10 * _N
_FLAT = 11 * _N + _L


def _pow20(x):
    x2 = x * x
    x4 = x2 * x2
    x8 = x4 * x4
    x16 = x8 * x8
    return x16 * x4


def _rsqrt_newton(r2):
    bits = lax.bitcast_convert_type(r2, jnp.int32)
    seed = jnp.int32(0x5F3759DF) - lax.shift_right_logical(bits, 1)
    y = lax.bitcast_convert_type(seed, jnp.float32)
    half = -0.5 * r2
    for _ in range(3):
        y = y * (1.5 + half * y * y)
    return y


def _sc_body(flat_hbm, rhor_hbm, rhoc_hbm, pairs_hbm, data, rho_v, rhoc, pair_v):
    wid = lax.axis_index("s") * 2 + lax.axis_index("c")

    pltpu.sync_copy(flat_hbm, data)

    def zero_body(k, _):
        rhoc[pl.ds(k * _L, _L)] = jnp.zeros((_L,), jnp.float32)
        return 0
    lax.fori_loop(0, _NJV, zero_body, 0)

    def _sload(off):
        return data[pl.ds(off, _L)][0]

    def row_body(il, pair_carry):
        i = jnp.where(il < _BAND, _BAND * wid + il,
                      _N - _BAND * (wid + 1) + (il - _BAND))
        xi = _sload(_OX + i)
        yi = _sload(_OY + i)
        zi = _sload(_OZ + i)
        ire_i = _sload(_ORE + i)
        be_i = _sload(_OBE + i)
        al_i = _sload(_OAL + i)
        fe_i = _sload(_OFE + i)
        a_i = _sload(_OA + i)
        bofe_i = _sload(_OB + i)
        ka_i = _sload(_OKA + i)
        la_i = _sload(_OLA + i)

        def visit(j0, pair_acc, rho_acc):
            xj = data[pl.ds(_OX + j0, _L)]
            yj = data[pl.ds(_OY + j0, _L)]
            zj = data[pl.ds(_OZ + j0, _L)]
            dx = xj - xi
            dy = yj - yi
            dz = zj - zi
            r2 = dx * dx + dy * dy + dz * dz
            r = r2 * _rsqrt_newton(r2)

            ire_j = data[pl.ds(_ORE + j0, _L)]
            be_j = data[pl.ds(_OBE + j0, _L)]
            al_j = data[pl.ds(_OAL + j0, _L)]
            fe_j = data[pl.ds(_OFE + j0, _L)]
            a_j = data[pl.ds(_OA + j0, _L)]
            bofe_j = data[pl.ds(_OB + j0, _L)]
            ka_j = data[pl.ds(_OKA + j0, _L)]
            la_j = data[pl.ds(_OLA + j0, _L)]

            u_i = r * ire_i
            om_i = 1.0 - u_i
            eb_i = jnp.exp(om_i * be_i)
            ea_i = jnp.exp(om_i * al_i)
            dlam_i = 1.0 + _pow20(u_i - la_i)
            idkap_i = 1.0 / (1.0 + _pow20(u_i - ka_i))
            fr_i = fe_i * eb_i / dlam_i
            phir_i = a_i * ea_i * idkap_i - bofe_i * fr_i

            u_j = r * ire_j
            om_j = 1.0 - u_j
            eb_j = jnp.exp(om_j * be_j)
            ea_j = jnp.exp(om_j * al_j)
            dlam_j = 1.0 + _pow20(u_j - la_j)
            idkap_j = 1.0 / (1.0 + _pow20(u_j - ka_j))
            fr_j = fe_j * eb_j / dlam_j
            phir_j = a_j * ea_j * idkap_j - bofe_j * fr_j

            q = fr_j / fr_i
            phi = q * phir_i + (fr_i / fr_j) * phir_j

            cols = j0 + lax.iota(jnp.int32, _L)
            tri = cols > i
            pmask = jnp.logical_and(tri, r <= _CUTOFF)
            pair_acc = pair_acc + jnp.where(pmask, phi, 0.0)
            rho_acc = rho_acc + jnp.where(tri, fr_j, 0.0)
            rhoc[pl.ds(j0, _L)] = (rhoc[pl.ds(j0, _L)]
                                   + jnp.where(tri, fr_i, 0.0))
            return pair_acc, rho_acc

        def col_block(blk, carry):
            pair_acc, rho_acc = carry
            j0 = blk * (4 * _L)
            for u in range(4):
                pair_acc, rho_acc = visit(j0 + u * _L, pair_acc, rho_acc)
            return pair_acc, rho_acc

        zero = jnp.zeros((_L,), jnp.float32)
        blk_lo = lax.div(i, 4 * _L)
        pair_acc, rho_acc = lax.fori_loop(blk_lo, _NJV // 4, col_block,
                                          (zero, zero))
        rho_v[pl.ds(il * _L, _L)] = rho_acc
        return pair_carry + pair_acc

    pair_tot = lax.fori_loop(0, _RPW, row_body, jnp.zeros((_L,), jnp.float32))
    pair_v[...] = pair_tot

    base_a = _BAND * wid
    base_b = _N - _BAND * (wid + 1)
    pltpu.sync_copy(rho_v.at[pl.ds(0, _BAND * _L)],
                    rhor_hbm.at[pl.ds(base_a * _L, _BAND * _L)])
    pltpu.sync_copy(rho_v.at[pl.ds(_BAND * _L, _BAND * _L)],
                    rhor_hbm.at[pl.ds(base_b * _L, _BAND * _L)])
    pltpu.sync_copy(rhoc, rhoc_hbm.at[wid])
    pltpu.sync_copy(pair_v, pairs_hbm.at[wid])


_sc_pairs = functools.partial(
    pl.kernel,
    out_type=(
        jax.ShapeDtypeStruct((_N * _L,), jnp.float32),
        jax.ShapeDtypeStruct((_NW, _N), jnp.float32),
        jax.ShapeDtypeStruct((_NW, _L), jnp.float32),
    ),
    mesh=plsc.VectorSubcoreMesh(core_axis_name="c", subcore_axis_name="s"),
    scratch_types=[
        pltpu.VMEM((_FLAT,), jnp.float32),
        pltpu.VMEM((_RPW * _L,), jnp.float32),
        pltpu.VMEM((_N,), jnp.float32),
        pltpu.VMEM((_L,), jnp.float32),
    ],
)(_sc_body)


def _tc_tail_kernel(rhor_ref, rhoc_ref, pt_ref, pairs_ref, out_ref):
    rho = (jnp.sum(rhor_ref[...], axis=0, keepdims=True)
           + jnp.sum(rhoc_ref[...], axis=0, keepdims=True))
    f_n0 = pt_ref[10:11, :]
    f_n1 = pt_ref[11:12, :]
    f_n2 = pt_ref[12:13, :]
    f_n3 = pt_ref[13:14, :]
    f_0 = pt_ref[14:15, :]
    f_1 = pt_ref[15:16, :]
    f_2 = pt_ref[16:17, :]
    f_3 = pt_ref[17:18, :]
    f_e = pt_ref[19:20, :]
    rho_n = pt_ref[20:21, :]
    rho_e = pt_ref[2:3, :]
    rho_0 = pt_ref[21:22, :]
    rho_s = pt_ref[3:4, :]
    eta = pt_ref[18:19, :]

    t_n = rho / rho_n - 1.0
    b1 = f_n0 + (f_n1 + (f_n2 + f_n3 * t_n) * t_n) * t_n
    t_e = rho / rho_e - 1.0
    b2 = f_0 + (f_1 + (f_2 + f_3 * t_e) * t_e) * t_e
    ratio = rho / rho_s
    lpw = eta * jnp.log(ratio)
    pw = jnp.exp(lpw)
    b3 = f_e * (1.0 - lpw) * pw
    f_val = jnp.where(rho < rho_n, b1, jnp.where(rho < rho_0, b2, b3))

    total = (jnp.sum(f_val, axis=(0, 1), keepdims=True)
             + 0.5 * jnp.sum(pairs_ref[...], axis=(0, 1), keepdims=True))
    out_ref[...] = total


def kernel(weights, params):
    x = weights[:, 0]
    y = weights[:, 1]
    z = weights[:, 2]
    flat = jnp.concatenate([
        x, y, z,
        1.0 / params[:, 0], params[:, 5], params[:, 4],
        params[:, 1], params[:, 6], params[:, 7] / params[:, 1],
        params[:, 8], params[:, 9],
        jnp.zeros((_L,), jnp.float32),
    ])
    rho_r, rho_c, pairs = _sc_pairs(flat)
    rho_rt = rho_r.reshape(_N, _L).T

    pt = params.T
    out = pl.pallas_call(
        _tc_tail_kernel,
        in_specs=[
            pl.BlockSpec((_L, _N), lambda: (0, 0)),
            pl.BlockSpec((_NW, _N), lambda: (0, 0)),
            pl.BlockSpec((22, _N), lambda: (0, 0)),
            pl.BlockSpec((_NW, _L), lambda: (0, 0)),
        ],
        out_specs=pl.BlockSpec((1, 1), lambda: (0, 0)),
        out_shape=jax.ShapeDtypeStruct((1, 1), jnp.float32),
    )(rho_rt, rho_c, pt, pairs)
    return out.reshape(())


# speedup vs baseline: 1.0271x; 1.0271x over previous
import functools

import jax
import jax.numpy as jnp
from jax import lax
from jax.experimental import pallas as pl
from jax.experimental.pallas import tpu as pltpu
from jax.experimental.pallas import tpu_sc as plsc

_N = 2048
_NW = 32
_BAND = _N // (2 * _NW)
_RPW = 2 * _BAND
_L = 16
_NJV = _N // _L
_CUTOFF = 5.0

_OX, _OY, _OZ = 0 * _N, 1 * _N, 2 * _N
_ORE, _OBE, _OAL = 3 * _N, 4 * _N, 5 * _N
_OFE, _OA, _OB = 6 * _N, 7 * _N, 8 * _N
_OKA, _OLA = 9 * _N, 10 * _N
_FLAT = 11 * _N + _L


def _pow20(x):
    x2 = x * x
    x4 = x2 * x2
    x8 = x4 * x4
    x16 = x8 * x8
    return x16 * x4


def _rsqrt_newton(r2):
    bits = lax.bitcast_convert_type(r2, jnp.int32)
    seed = jnp.int32(0x5F3759DF) - lax.shift_right_logical(bits, 1)
    y = lax.bitcast_convert_type(seed, jnp.float32)
    half = -0.5 * r2
    for _ in range(3):
        y = y * (1.5 + half * y * y)
    return y


def _sc_body(flat_hbm, rhor_hbm, rhoc_hbm, pairs_hbm, data, rho_v, rhoc, pair_v):
    wid = lax.axis_index("s") * 2 + lax.axis_index("c")

    pltpu.sync_copy(flat_hbm, data)

    def zero_body(k, _):
        rhoc[pl.ds(k * _L, _L)] = jnp.zeros((_L,), jnp.float32)
        return 0
    lax.fori_loop(0, _NJV, zero_body, 0)

    def _sload(off):
        return data[pl.ds(off, _L)][0]

    def row_body(il, pair_carry):
        i = jnp.where(il < _BAND, _BAND * wid + il,
                      _N - _BAND * (wid + 1) + (il - _BAND))
        xi = _sload(_OX + i)
        yi = _sload(_OY + i)
        zi = _sload(_OZ + i)
        ire_i = _sload(_ORE + i)
        be_i = _sload(_OBE + i)
        al_i = _sload(_OAL + i)
        fe_i = _sload(_OFE + i)
        a_i = _sload(_OA + i)
        bofe_i = _sload(_OB + i)
        ka_i = _sload(_OKA + i)
        la_i = _sload(_OLA + i)

        def visit(j0, pair_acc, rho_acc):
            xj = data[pl.ds(_OX + j0, _L)]
            yj = data[pl.ds(_OY + j0, _L)]
            zj = data[pl.ds(_OZ + j0, _L)]
            dx = xj - xi
            dy = yj - yi
            dz = zj - zi
            r2 = dx * dx + dy * dy + dz * dz
            r = r2 * _rsqrt_newton(r2)

            ire_j = data[pl.ds(_ORE + j0, _L)]
            be_j = data[pl.ds(_OBE + j0, _L)]
            al_j = data[pl.ds(_OAL + j0, _L)]
            fe_j = data[pl.ds(_OFE + j0, _L)]
            a_j = data[pl.ds(_OA + j0, _L)]
            bofe_j = data[pl.ds(_OB + j0, _L)]
            ka_j = data[pl.ds(_OKA + j0, _L)]
            la_j = data[pl.ds(_OLA + j0, _L)]

            u_i = r * ire_i
            om_i = 1.0 - u_i
            eb_i = jnp.exp(om_i * be_i)
            ea_i = jnp.exp(om_i * al_i)
            dlam_i = 1.0 + _pow20(u_i - la_i)
            idkap_i = 1.0 / (1.0 + _pow20(u_i - ka_i))
            fr_i = fe_i * eb_i / dlam_i
            phir_i = a_i * ea_i * idkap_i - bofe_i * fr_i

            u_j = r * ire_j
            om_j = 1.0 - u_j
            eb_j = jnp.exp(om_j * be_j)
            ea_j = jnp.exp(om_j * al_j)
            dlam_j = 1.0 + _pow20(u_j - la_j)
            idkap_j = 1.0 / (1.0 + _pow20(u_j - ka_j))
            fr_j = fe_j * eb_j / dlam_j
            phir_j = a_j * ea_j * idkap_j - bofe_j * fr_j

            q = fr_j / fr_i
            phi = q * phir_i + (fr_i / fr_j) * phir_j

            cols = j0 + lax.iota(jnp.int32, _L)
            tri = cols > i
            pmask = jnp.logical_and(tri, r <= _CUTOFF)
            pair_acc = pair_acc + jnp.where(pmask, phi, 0.0)
            rho_acc = rho_acc + jnp.where(tri, fr_j, 0.0)
            rhoc[pl.ds(j0, _L)] = (rhoc[pl.ds(j0, _L)]
                                   + jnp.where(tri, fr_i, 0.0))
            return pair_acc, rho_acc

        zero = jnp.zeros((_L,), jnp.float32)
        jv_lo = lax.div(i, _L)
        pair_acc, rho_acc = plsc.parallel_loop(
            jv_lo, _NJV, unroll=4, carry=(zero, zero),
        )(lambda jv, carry: visit(jv * _L, carry[0], carry[1]))
        rho_v[pl.ds(il * _L, _L)] = rho_acc
        return pair_carry + pair_acc

    pair_tot = lax.fori_loop(0, _RPW, row_body, jnp.zeros((_L,), jnp.float32))
    pair_v[...] = pair_tot

    base_a = _BAND * wid
    base_b = _N - _BAND * (wid + 1)
    pltpu.sync_copy(rho_v.at[pl.ds(0, _BAND * _L)],
                    rhor_hbm.at[pl.ds(base_a * _L, _BAND * _L)])
    pltpu.sync_copy(rho_v.at[pl.ds(_BAND * _L, _BAND * _L)],
                    rhor_hbm.at[pl.ds(base_b * _L, _BAND * _L)])
    pltpu.sync_copy(rhoc, rhoc_hbm.at[wid])
    pltpu.sync_copy(pair_v, pairs_hbm.at[wid])


_sc_pairs = functools.partial(
    pl.kernel,
    out_type=(
        jax.ShapeDtypeStruct((_N * _L,), jnp.float32),
        jax.ShapeDtypeStruct((_NW, _N), jnp.float32),
        jax.ShapeDtypeStruct((_NW, _L), jnp.float32),
    ),
    mesh=plsc.VectorSubcoreMesh(core_axis_name="c", subcore_axis_name="s"),
    scratch_types=[
        pltpu.VMEM((_FLAT,), jnp.float32),
        pltpu.VMEM((_RPW * _L,), jnp.float32),
        pltpu.VMEM((_N,), jnp.float32),
        pltpu.VMEM((_L,), jnp.float32),
    ],
)(_sc_body)


def _tc_tail_kernel(rhor_ref, rhoc_ref, pt_ref, pairs_ref, out_ref):
    rho = (jnp.sum(rhor_ref[...], axis=0, keepdims=True)
           + jnp.sum(rhoc_ref[...], axis=0, keepdims=True))
    f_n0 = pt_ref[10:11, :]
    f_n1 = pt_ref[11:12, :]
    f_n2 = pt_ref[12:13, :]
    f_n3 = pt_ref[13:14, :]
    f_0 = pt_ref[14:15, :]
    f_1 = pt_ref[15:16, :]
    f_2 = pt_ref[16:17, :]
    f_3 = pt_ref[17:18, :]
    f_e = pt_ref[19:20, :]
    rho_n = pt_ref[20:21, :]
    rho_e = pt_ref[2:3, :]
    rho_0 = pt_ref[21:22, :]
    rho_s = pt_ref[3:4, :]
    eta = pt_ref[18:19, :]

    t_n = rho / rho_n - 1.0
    b1 = f_n0 + (f_n1 + (f_n2 + f_n3 * t_n) * t_n) * t_n
    t_e = rho / rho_e - 1.0
    b2 = f_0 + (f_1 + (f_2 + f_3 * t_e) * t_e) * t_e
    ratio = rho / rho_s
    lpw = eta * jnp.log(ratio)
    pw = jnp.exp(lpw)
    b3 = f_e * (1.0 - lpw) * pw
    f_val = jnp.where(rho < rho_n, b1, jnp.where(rho < rho_0, b2, b3))

    total = (jnp.sum(f_val, axis=(0, 1), keepdims=True)
             + 0.5 * jnp.sum(pairs_ref[...], axis=(0, 1), keepdims=True))
    out_ref[...] = total


def kernel(weights, params):
    x = weights[:, 0]
    y = weights[:, 1]
    z = weights[:, 2]
    flat = jnp.concatenate([
        x, y, z,
        1.0 / params[:, 0], params[:, 5], params[:, 4],
        params[:, 1], params[:, 6], params[:, 7] / params[:, 1],
        params[:, 8], params[:, 9],
        jnp.zeros((_L,), jnp.float32),
    ])
    rho_r, rho_c, pairs = _sc_pairs(flat)
    rho_rt = rho_r.reshape(_N, _L).T

    pt = params.T
    out = pl.pallas_call(
        _tc_tail_kernel,
        in_specs=[
            pl.BlockSpec((_L, _N), lambda: (0, 0)),
            pl.BlockSpec((_NW, _N), lambda: (0, 0)),
            pl.BlockSpec((22, _N), lambda: (0, 0)),
            pl.BlockSpec((_NW, _L), lambda: (0, 0)),
        ],
        out_specs=pl.BlockSpec((1, 1), lambda: (0, 0)),
        out_shape=jax.ShapeDtypeStruct((1, 1), jnp.float32),
    )(rho_rt, rho_c, pt, pairs)
    return out.reshape(())


# speedup vs baseline: 1.0522x; 1.0245x over previous
import functools

import jax
import jax.numpy as jnp
from jax import lax
from jax.experimental import pallas as pl
from jax.experimental.pallas import tpu as pltpu
from jax.experimental.pallas import tpu_sc as plsc

_N = 2048
_NW = 32
_BAND = _N // (2 * _NW)
_RPW = 2 * _BAND
_L = 16
_NJV = _N // _L
_CUTOFF = 5.0

_OX, _OY, _OZ = 0 * _N, 1 * _N, 2 * _N
_ORE, _OBE, _OAL = 3 * _N, 4 * _N, 5 * _N
_OFE, _OA, _OB = 6 * _N, 7 * _N, 8 * _N
_OKA, _OLA = 9 * _N, 10 * _N
_FLAT = 11 * _N + _L


def _pow20(x):
    x2 = x * x
    x4 = x2 * x2
    x8 = x4 * x4
    x16 = x8 * x8
    return x16 * x4


def _rsqrt_newton(r2):
    bits = lax.bitcast_convert_type(r2, jnp.int32)
    seed = jnp.int32(0x5F3759DF) - lax.shift_right_logical(bits, 1)
    y = lax.bitcast_convert_type(seed, jnp.float32)
    half = -0.5 * r2
    for _ in range(3):
        y = y * (1.5 + half * y * y)
    return y


def _sc_body(flat_hbm, rhor_hbm, rhoc_hbm, pairs_hbm, data, rho_v, rhoc, pair_v):
    wid = lax.axis_index("s") * 2 + lax.axis_index("c")

    pltpu.sync_copy(flat_hbm, data)

    def zero_body(k, _):
        rhoc[pl.ds(k * _L, _L)] = jnp.zeros((_L,), jnp.float32)
        return 0
    lax.fori_loop(0, _NJV, zero_body, 0)

    def _sload(off):
        return data[pl.ds(off, _L)][0]

    def row_body(il, pair_carry):
        i = jnp.where(il < _BAND, _BAND * wid + il,
                      _N - _BAND * (wid + 1) + (il - _BAND))
        xi = _sload(_OX + i)
        yi = _sload(_OY + i)
        zi = _sload(_OZ + i)
        ire_i = _sload(_ORE + i)
        be_i = _sload(_OBE + i)
        al_i = _sload(_OAL + i)
        fe_i = _sload(_OFE + i)
        a_i = _sload(_OA + i)
        bofe_i = _sload(_OB + i)
        ka_i = _sload(_OKA + i)
        la_i = _sload(_OLA + i)

        def visit(j0, pair_acc, rho_acc):
            xj = data[pl.ds(_OX + j0, _L)]
            yj = data[pl.ds(_OY + j0, _L)]
            zj = data[pl.ds(_OZ + j0, _L)]
            dx = xj - xi
            dy = yj - yi
            dz = zj - zi
            r2 = dx * dx + dy * dy + dz * dz
            r = r2 * _rsqrt_newton(r2)

            ire_j = data[pl.ds(_ORE + j0, _L)]
            be_j = data[pl.ds(_OBE + j0, _L)]
            al_j = data[pl.ds(_OAL + j0, _L)]
            fe_j = data[pl.ds(_OFE + j0, _L)]
            a_j = data[pl.ds(_OA + j0, _L)]
            bofe_j = data[pl.ds(_OB + j0, _L)]
            ka_j = data[pl.ds(_OKA + j0, _L)]
            la_j = data[pl.ds(_OLA + j0, _L)]

            u_i = r * ire_i
            om_i = 1.0 - u_i
            eb_i = jnp.exp(om_i * be_i)
            ea_i = jnp.exp(om_i * al_i)
            dlam_i = 1.0 + _pow20(u_i - la_i)
            idkap_i = 1.0 / (1.0 + _pow20(u_i - ka_i))
            fr_i = fe_i * eb_i / dlam_i
            phir_i = a_i * ea_i * idkap_i - bofe_i * fr_i

            u_j = r * ire_j
            om_j = 1.0 - u_j
            eb_j = jnp.exp(om_j * be_j)
            ea_j = jnp.exp(om_j * al_j)
            dlam_j = 1.0 + _pow20(u_j - la_j)
            idkap_j = 1.0 / (1.0 + _pow20(u_j - ka_j))
            fr_j = fe_j * eb_j / dlam_j
            phir_j = a_j * ea_j * idkap_j - bofe_j * fr_j

            q = fr_j / fr_i
            phi = q * phir_i + (fr_i / fr_j) * phir_j

            cols = j0 + lax.iota(jnp.int32, _L)
            tri = cols > i
            pmask = jnp.logical_and(tri, r <= _CUTOFF)
            pair_acc = pair_acc + jnp.where(pmask, phi, 0.0)
            rho_acc = rho_acc + jnp.where(tri, fr_j, 0.0)
            plsc.addupdate(rhoc.at[pl.ds(j0, _L)], jnp.where(tri, fr_i, 0.0))
            return pair_acc, rho_acc

        zero = jnp.zeros((_L,), jnp.float32)
        jv_lo = lax.div(i, _L)
        pair_acc, rho_acc = plsc.parallel_loop(
            jv_lo, _NJV, unroll=4, carry=(zero, zero),
        )(lambda jv, carry: visit(jv * _L, carry[0], carry[1]))
        rho_v[pl.ds(il * _L, _L)] = rho_acc
        return pair_carry + pair_acc

    pair_tot = lax.fori_loop(0, _RPW, row_body, jnp.zeros((_L,), jnp.float32))
    pair_v[...] = pair_tot

    base_a = _BAND * wid
    base_b = _N - _BAND * (wid + 1)
    pltpu.sync_copy(rho_v.at[pl.ds(0, _BAND * _L)],
                    rhor_hbm.at[pl.ds(base_a * _L, _BAND * _L)])
    pltpu.sync_copy(rho_v.at[pl.ds(_BAND * _L, _BAND * _L)],
                    rhor_hbm.at[pl.ds(base_b * _L, _BAND * _L)])
    pltpu.sync_copy(rhoc, rhoc_hbm.at[wid])
    pltpu.sync_copy(pair_v, pairs_hbm.at[wid])


_sc_pairs = functools.partial(
    pl.kernel,
    out_type=(
        jax.ShapeDtypeStruct((_N * _L,), jnp.float32),
        jax.ShapeDtypeStruct((_NW, _N), jnp.float32),
        jax.ShapeDtypeStruct((_NW, _L), jnp.float32),
    ),
    mesh=plsc.VectorSubcoreMesh(core_axis_name="c", subcore_axis_name="s"),
    scratch_types=[
        pltpu.VMEM((_FLAT,), jnp.float32),
        pltpu.VMEM((_RPW * _L,), jnp.float32),
        pltpu.VMEM((_N,), jnp.float32),
        pltpu.VMEM((_L,), jnp.float32),
    ],
)(_sc_body)


def _tc_tail_kernel(rhor_ref, rhoc_ref, pt_ref, pairs_ref, out_ref):
    rho = (jnp.sum(rhor_ref[...], axis=0, keepdims=True)
           + jnp.sum(rhoc_ref[...], axis=0, keepdims=True))
    f_n0 = pt_ref[10:11, :]
    f_n1 = pt_ref[11:12, :]
    f_n2 = pt_ref[12:13, :]
    f_n3 = pt_ref[13:14, :]
    f_0 = pt_ref[14:15, :]
    f_1 = pt_ref[15:16, :]
    f_2 = pt_ref[16:17, :]
    f_3 = pt_ref[17:18, :]
    f_e = pt_ref[19:20, :]
    rho_n = pt_ref[20:21, :]
    rho_e = pt_ref[2:3, :]
    rho_0 = pt_ref[21:22, :]
    rho_s = pt_ref[3:4, :]
    eta = pt_ref[18:19, :]

    t_n = rho / rho_n - 1.0
    b1 = f_n0 + (f_n1 + (f_n2 + f_n3 * t_n) * t_n) * t_n
    t_e = rho / rho_e - 1.0
    b2 = f_0 + (f_1 + (f_2 + f_3 * t_e) * t_e) * t_e
    ratio = rho / rho_s
    lpw = eta * jnp.log(ratio)
    pw = jnp.exp(lpw)
    b3 = f_e * (1.0 - lpw) * pw
    f_val = jnp.where(rho < rho_n, b1, jnp.where(rho < rho_0, b2, b3))

    total = (jnp.sum(f_val, axis=(0, 1), keepdims=True)
             + 0.5 * jnp.sum(pairs_ref[...], axis=(0, 1), keepdims=True))
    out_ref[...] = total


def kernel(weights, params):
    x = weights[:, 0]
    y = weights[:, 1]
    z = weights[:, 2]
    flat = jnp.concatenate([
        x, y, z,
        1.0 / params[:, 0], params[:, 5], params[:, 4],
        params[:, 1], params[:, 6], params[:, 7] / params[:, 1],
        params[:, 8], params[:, 9],
        jnp.zeros((_L,), jnp.float32),
    ])
    rho_r, rho_c, pairs = _sc_pairs(flat)
    rho_rt = rho_r.reshape(_N, _L).T

    pt = params.T
    out = pl.pallas_call(
        _tc_tail_kernel,
        in_specs=[
            pl.BlockSpec((_L, _N), lambda: (0, 0)),
            pl.BlockSpec((_NW, _N), lambda: (0, 0)),
            pl.BlockSpec((22, _N), lambda: (0, 0)),
            pl.BlockSpec((_NW, _L), lambda: (0, 0)),
        ],
        out_specs=pl.BlockSpec((1, 1), lambda: (0, 0)),
        out_shape=jax.ShapeDtypeStruct((1, 1), jnp.float32),
    )(rho_rt, rho_c, pt, pairs)
    return out.reshape(())
